# baseline (speedup 1.0000x reference)
import jax
import jax.numpy as jnp
from jax.experimental import pallas as pl
from jax.experimental.pallas import tpu as pltpu

CLS_NUM = 1000
D = 512
N_PREFIX = 3
N_CLS_TOK = 3
CTX_LEN = 77
PAD_SIZE = 67
PAD_LEN = 75

B = 40
NSTEPS = CLS_NUM // B
ROWS = B * CTX_LEN


def _body(ctx_ref, sot_ref, eot_ref, pad_ref, cls_ref, out_ref, t0, tmpl):
    @pl.when(pl.program_id(0) == 0)
    def _build():
        t0[0:1, :] = sot_ref[0, :, :]
        t0[1:4, :] = ctx_ref[0, 0:3, :]
        t0[4:7, :] = ctx_ref[0, 0:3, :]
        t0[7:9, :] = ctx_ref[0, 3:, :]
        t0[9:10, :] = eot_ref[0, :, :]
        t0[10:, :] = pad_ref[0, 0:PAD_SIZE, :]
        tmpl[:, :] = jnp.broadcast_to(t0[:, :][None], (B, CTX_LEN, D)).reshape(
            ROWS, D
        )

    out_ref[:, :] = tmpl[:, :]
    for c in range(B):
        out_ref[c * CTX_LEN + 4 : c * CTX_LEN + 7, :] = cls_ref[c, :, :]


def kernel(ctx, emb_sot, emb_cls, emb_eot, emb_pad):
    out2d = pl.pallas_call(
        _body,
        grid=(NSTEPS,),
        in_specs=[
            pl.BlockSpec((1, 5, D), lambda i: (0, 0, 0)),
            pl.BlockSpec((1, 1, D), lambda i: (0, 0, 0)),
            pl.BlockSpec((1, 1, D), lambda i: (0, 0, 0)),
            pl.BlockSpec((1, PAD_LEN, D), lambda i: (0, 0, 0)),
            pl.BlockSpec((B, N_CLS_TOK, D), lambda i: (i, 0, 0)),
        ],
        out_specs=pl.BlockSpec((ROWS, D), lambda i: (i, 0)),
        out_shape=jax.ShapeDtypeStruct((CLS_NUM * CTX_LEN, D), jnp.float32),
        scratch_shapes=[
            pltpu.VMEM((CTX_LEN, D), jnp.float32),
            pltpu.VMEM((ROWS, D), jnp.float32),
        ],
    )(ctx, emb_sot, emb_eot, emb_pad, emb_cls)
    return out2d.reshape(CLS_NUM, CTX_LEN, D)


# speedup vs baseline: 1.1638x; 1.1638x over previous
import functools

import jax
import jax.numpy as jnp
from jax import lax
from jax.experimental import pallas as pl
from jax.experimental.pallas import tpu as pltpu
from jax.experimental.pallas import tpu_sc as plsc

CLS_NUM = 1000
D = 512
N_PREFIX = 3
N_SUFFIX = 2
N_CTX = N_PREFIX + N_SUFFIX
N_CLS_TOK = 3
CTX_LEN = 77
PAD_SIZE = CTX_LEN - (N_CTX + N_CLS_TOK + 2)
PAD_LEN = 75
HEAD = 8
TAIL = CTX_LEN - HEAD

NC = 2
NS = 16
NW = NC * NS
MAX_PER_W = (CLS_NUM + NW - 1) // NW

B = 125
NSTEPS = CLS_NUM // B


def _tc_body(ctx_ref, sot_ref, eot_ref, pad_ref, cls_ref, head_ref, tail_ref):
    b = head_ref.shape[0]
    head_ref[:, 0:1, :] = jnp.broadcast_to(sot_ref[0, :, :][None], (b, 1, D))
    head_ref[:, 1:4, :] = jnp.broadcast_to(
        ctx_ref[0, 0:N_PREFIX, :][None], (b, N_PREFIX, D))
    head_ref[:, 4:7, :] = cls_ref[:, :, :]
    head_ref[:, 7:8, :] = jnp.broadcast_to(
        ctx_ref[0, N_PREFIX:N_PREFIX + 1, :][None], (b, 1, D))
    tail_ref[0:1, :] = ctx_ref[0, N_PREFIX + 1:, :]
    tail_ref[1:2, :] = eot_ref[0, :, :]
    tail_ref[2:, :] = pad_ref[0, 0:PAD_SIZE, :]


def kernel(ctx, emb_sot, emb_cls, emb_eot, emb_pad):
    head8, tail69 = pl.pallas_call(
        _tc_body,
        grid=(NSTEPS,),
        in_specs=[
            pl.BlockSpec((1, N_CTX, D), lambda i: (0, 0, 0)),
            pl.BlockSpec((1, 1, D), lambda i: (0, 0, 0)),
            pl.BlockSpec((1, 1, D), lambda i: (0, 0, 0)),
            pl.BlockSpec((1, PAD_LEN, D), lambda i: (0, 0, 0)),
            pl.BlockSpec((B, N_CLS_TOK, D), lambda i: (i, 0, 0)),
        ],
        out_specs=[
            pl.BlockSpec((B, HEAD, D), lambda i: (i, 0, 0)),
            pl.BlockSpec((TAIL, D), lambda i: (0, 0)),
        ],
        out_shape=[
            jax.ShapeDtypeStruct((CLS_NUM, HEAD, D), jnp.float32),
            jax.ShapeDtypeStruct((TAIL, D), jnp.float32),
        ],
    )(ctx, emb_sot, emb_eot, emb_pad, emb_cls)

    mesh = plsc.VectorSubcoreMesh(core_axis_name="c", subcore_axis_name="s")

    @functools.partial(
        pl.kernel,
        mesh=mesh,
        out_type=jax.ShapeDtypeStruct((CLS_NUM, CTX_LEN, D), jnp.float32),
        scratch_types=[
            pltpu.VMEM((TAIL, D), jnp.float32),
            pltpu.SemaphoreType.DMA,
        ],
    )
    def sc_kernel(head_hbm, tail_hbm, out_hbm, tmpl, sem):
        cid = lax.axis_index("c")
        sid = lax.axis_index("s")
        wid = sid * NC + cid

        pltpu.sync_copy(tail_hbm, tmpl)

        def class_copies(c):
            return (
                pltpu.make_async_copy(
                    head_hbm.at[c], out_hbm.at[c, pl.ds(0, HEAD)], sem),
                pltpu.make_async_copy(
                    tmpl, out_hbm.at[c, pl.ds(HEAD, TAIL)], sem),
            )

        @pl.loop(0, MAX_PER_W)
        def _issue(i):
            c = wid + i * NW

            @pl.when(c < CLS_NUM)
            def _():
                for cp in class_copies(c):
                    cp.start()

        @pl.loop(0, MAX_PER_W)
        def _drain(i):
            c = wid + i * NW

            @pl.when(c < CLS_NUM)
            def _():
                for cp in class_copies(c):
                    cp.wait()

    return sc_kernel(head8, tail69)


# speedup vs baseline: 2.4065x; 2.0678x over previous
import jax
import jax.numpy as jnp
from jax.experimental import pallas as pl
from jax.experimental.pallas import tpu as pltpu

CLS_NUM = 1000
D = 512
N_PREFIX = 3
N_SUFFIX = 2
N_CTX = N_PREFIX + N_SUFFIX
N_CLS_TOK = 3
CTX_LEN = 77
PAD_SIZE = CTX_LEN - (N_CTX + N_CLS_TOK + 2)
PAD_LEN = 75

NG = 10
G = CLS_NUM // NG
NPRIO = 2


def _body(ctx_ref, sot_ref, eot_ref, pad_ref, cls_hbm, out_hbm,
          t0, tmpl, tmpl_sem, cls_sem):
    t0[0:1, :] = sot_ref[0, :, :]
    t0[1:4, :] = ctx_ref[0, 0:N_PREFIX, :]
    t0[4:7, :] = ctx_ref[0, 0:N_CLS_TOK, :]
    t0[7:9, :] = ctx_ref[0, N_PREFIX:, :]
    t0[9:10, :] = eot_ref[0, :, :]
    t0[10:, :] = pad_ref[0, 0:PAD_SIZE, :]
    tmpl[:, :, :] = jnp.broadcast_to(t0[:, :][None], (G, CTX_LEN, D))

    for g in range(NG):
        pltpu.async_copy(
            tmpl, out_hbm.at[pl.ds(g * G, G)], tmpl_sem.at[g],
            priority=g % NPRIO)
    for g in range(NG):
        pltpu.make_async_copy(
            tmpl, out_hbm.at[pl.ds(g * G, G)], tmpl_sem.at[g]).wait()
        pltpu.async_copy(
            cls_hbm.at[pl.ds(g * G, G)],
            out_hbm.at[pl.ds(g * G, G), pl.ds(4, N_CLS_TOK)],
            cls_sem.at[g], priority=g % NPRIO)
    for g in range(NG):
        pltpu.make_async_copy(
            cls_hbm.at[pl.ds(g * G, G)],
            out_hbm.at[pl.ds(g * G, G), pl.ds(4, N_CLS_TOK)],
            cls_sem.at[g]).wait()


def kernel(ctx, emb_sot, emb_cls, emb_eot, emb_pad):
    return pl.pallas_call(
        _body,
        in_specs=[
            pl.BlockSpec(memory_space=pltpu.MemorySpace.VMEM),
            pl.BlockSpec(memory_space=pltpu.MemorySpace.VMEM),
            pl.BlockSpec(memory_space=pltpu.MemorySpace.VMEM),
            pl.BlockSpec(memory_space=pltpu.MemorySpace.VMEM),
            pl.BlockSpec(memory_space=pltpu.MemorySpace.HBM),
        ],
        out_specs=pl.BlockSpec(memory_space=pltpu.MemorySpace.HBM),
        out_shape=jax.ShapeDtypeStruct((CLS_NUM, CTX_LEN, D), jnp.float32),
        scratch_shapes=[
            pltpu.VMEM((CTX_LEN, D), jnp.float32),
            pltpu.VMEM((G, CTX_LEN, D), jnp.float32),
            pltpu.SemaphoreType.DMA((NG,)),
            pltpu.SemaphoreType.DMA((NG,)),
        ],
    )(ctx, emb_sot, emb_eot, emb_pad, emb_cls)


# speedup vs baseline: 4.3243x; 1.7969x over previous
import jax
import jax.numpy as jnp
from jax.experimental import pallas as pl
from jax.experimental.pallas import tpu as pltpu

CLS_NUM = 1000
D = 512
N_PREFIX = 3
N_SUFFIX = 2
N_CTX = N_PREFIX + N_SUFFIX
N_CLS_TOK = 3
CTX_LEN = 77
PAD_SIZE = CTX_LEN - (N_CTX + N_CLS_TOK + 2)
PAD_LEN = 75

NG = 10
G = CLS_NUM // NG
SPLIT = 72


def _body(ctx_ref, sot_ref, eot_ref, pad_ref, cls_ref, out_hbm,
          t0, bufs, sems):
    i = pl.program_id(0)

    @pl.when(i == 0)
    def _build():
        t0[0:1, :] = sot_ref[0, :, :]
        t0[1:4, :] = ctx_ref[0, 0:N_PREFIX, :]
        t0[4:7, :] = ctx_ref[0, 0:N_CLS_TOK, :]
        t0[7:9, :] = ctx_ref[0, N_PREFIX:, :]
        t0[9:10, :] = eot_ref[0, :, :]
        t0[10:, :] = pad_ref[0, 0:PAD_SIZE, :]
        rep = jnp.broadcast_to(t0[:, :][None], (G, CTX_LEN, D))
        bufs[0, :, :, :] = rep
        bufs[1, :, :, :] = rep

    def copies(step, j):
        return (
            pltpu.make_async_copy(
                bufs.at[j, :, pl.ds(0, SPLIT)],
                out_hbm.at[pl.ds(step * G, G), pl.ds(0, SPLIT)],
                sems.at[j, 0]),
            pltpu.make_async_copy(
                bufs.at[j, :, pl.ds(SPLIT, CTX_LEN - SPLIT)],
                out_hbm.at[pl.ds(step * G, G), pl.ds(SPLIT, CTX_LEN - SPLIT)],
                sems.at[j, 1]),
        )

    for j in range(2):
        @pl.when(jnp.logical_and(i >= 2, i % 2 == j))
        def _wait_prev():
            for cp in copies(i - 2, j):
                cp.wait()

    for j in range(2):
        @pl.when(i % 2 == j)
        def _emit():
            bufs[j, :, 4:7, :] = cls_ref[:, :, :]
            for cp in copies(i, j):
                cp.start()

    @pl.when(i == NG - 1)
    def _drain():
        for cp in copies(NG - 2, (NG - 2) % 2):
            cp.wait()
        for cp in copies(NG - 1, (NG - 1) % 2):
            cp.wait()


def kernel(ctx, emb_sot, emb_cls, emb_eot, emb_pad):
    return pl.pallas_call(
        _body,
        grid=(NG,),
        in_specs=[
            pl.BlockSpec((1, N_CTX, D), lambda i: (0, 0, 0)),
            pl.BlockSpec((1, 1, D), lambda i: (0, 0, 0)),
            pl.BlockSpec((1, 1, D), lambda i: (0, 0, 0)),
            pl.BlockSpec((1, PAD_LEN, D), lambda i: (0, 0, 0)),
            pl.BlockSpec((G, N_CLS_TOK, D), lambda i: (i, 0, 0)),
        ],
        out_specs=pl.BlockSpec(memory_space=pltpu.MemorySpace.HBM),
        out_shape=jax.ShapeDtypeStruct((CLS_NUM, CTX_LEN, D), jnp.float32),
        scratch_shapes=[
            pltpu.VMEM((CTX_LEN, D), jnp.float32),
            pltpu.VMEM((2, G, CTX_LEN, D), jnp.float32),
            pltpu.SemaphoreType.DMA((2, 2)),
        ],
    )(ctx, emb_sot, emb_eot, emb_pad, emb_cls)


# speedup vs baseline: 4.3389x; 1.0034x over previous
import jax
import jax.numpy as jnp
from jax.experimental import pallas as pl
from jax.experimental.pallas import tpu as pltpu

CLS_NUM = 1000
D = 512
N_PREFIX = 3
N_SUFFIX = 2
N_CTX = N_PREFIX + N_SUFFIX
N_CLS_TOK = 3
CTX_LEN = 77
PAD_SIZE = CTX_LEN - (N_CTX + N_CLS_TOK + 2)
PAD_LEN = 75

NG = 10
G = CLS_NUM // NG
SPLIT = 72


def _body(ctx_ref, sot_ref, eot_ref, pad_ref, cls_ref, out_hbm,
          t0, bufs, sems):
    i = pl.program_id(0)

    @pl.when(i == 0)
    def _build():
        t0[0:1, :] = sot_ref[0, :, :]
        t0[1:4, :] = ctx_ref[0, 0:N_PREFIX, :]
        t0[4:7, :] = ctx_ref[0, 0:N_CLS_TOK, :]
        t0[7:9, :] = ctx_ref[0, N_PREFIX:, :]
        t0[9:10, :] = eot_ref[0, :, :]
        t0[10:, :] = pad_ref[0, 0:PAD_SIZE, :]
        rep = jnp.broadcast_to(t0[:, :][None], (G, CTX_LEN, D))
        bufs[0, :, :, :] = rep
        bufs[1, :, :, :] = rep

    def copies(step, j):
        per_class = [
            pltpu.make_async_copy(
                bufs.at[j, k, pl.ds(0, SPLIT)],
                out_hbm.at[step * G + k, pl.ds(0, SPLIT)],
                sems.at[j, 0])
            for k in range(G)
        ]
        per_class.append(
            pltpu.make_async_copy(
                bufs.at[j, :, pl.ds(SPLIT, CTX_LEN - SPLIT)],
                out_hbm.at[pl.ds(step * G, G), pl.ds(SPLIT, CTX_LEN - SPLIT)],
                sems.at[j, 1]))
        return per_class

    for j in range(2):
        @pl.when(jnp.logical_and(i >= 2, i % 2 == j))
        def _wait_prev():
            for cp in copies(i - 2, j):
                cp.wait()

    for j in range(2):
        @pl.when(i % 2 == j)
        def _emit():
            bufs[j, :, 4:7, :] = cls_ref[:, :, :]
            for cp in copies(i, j):
                cp.start()

    @pl.when(i == NG - 1)
    def _drain():
        for cp in copies(NG - 2, (NG - 2) % 2):
            cp.wait()
        for cp in copies(NG - 1, (NG - 1) % 2):
            cp.wait()


def kernel(ctx, emb_sot, emb_cls, emb_eot, emb_pad):
    return pl.pallas_call(
        _body,
        grid=(NG,),
        in_specs=[
            pl.BlockSpec((1, N_CTX, D), lambda i: (0, 0, 0)),
            pl.BlockSpec((1, 1, D), lambda i: (0, 0, 0)),
            pl.BlockSpec((1, 1, D), lambda i: (0, 0, 0)),
            pl.BlockSpec((1, PAD_LEN, D), lambda i: (0, 0, 0)),
            pl.BlockSpec((G, N_CLS_TOK, D), lambda i: (i, 0, 0)),
        ],
        out_specs=pl.BlockSpec(memory_space=pltpu.MemorySpace.HBM),
        out_shape=jax.ShapeDtypeStruct((CLS_NUM, CTX_LEN, D), jnp.float32),
        scratch_shapes=[
            pltpu.VMEM((CTX_LEN, D), jnp.float32),
            pltpu.VMEM((2, G, CTX_LEN, D), jnp.float32),
            pltpu.SemaphoreType.DMA((2, 2)),
        ],
    )(ctx, emb_sot, emb_eot, emb_pad, emb_cls)


# speedup vs baseline: 9.5894x; 2.2101x over previous
import jax
import jax.numpy as jnp
from jax.experimental import pallas as pl

CLS_NUM = 1000
D = 512
N_PREFIX = 3
N_SUFFIX = 2
N_CTX = N_PREFIX + N_SUFFIX
N_CLS_TOK = 3
CTX_LEN = 77
PAD_SIZE = CTX_LEN - (N_CTX + N_CLS_TOK + 2)
PAD_LEN = 75


def _body(ctx_ref, sot_ref, eot_ref, pad_ref, cls_ref, out_ref):
    t = pl.program_id(0)

    def bcast(row):
        out_ref[0, :, :] = jnp.broadcast_to(row.reshape(1, D), (CLS_NUM, D))

    @pl.when(t == 0)
    def _():
        bcast(sot_ref[0, 0:1, :])

    @pl.when(jnp.logical_and(t >= 1, t < 4))
    def _():
        bcast(ctx_ref[0, pl.ds(jnp.clip(t - 1, 0, N_CTX - 1), 1), :])

    @pl.when(jnp.logical_and(t >= 4, t < 7))
    def _():
        j = jnp.clip(t - 4, 0, N_CLS_TOK - 1)
        out_ref[0, :, :] = cls_ref[:, pl.ds(j, 1), :].reshape(CLS_NUM, D)

    @pl.when(jnp.logical_and(t >= 7, t < 9))
    def _():
        bcast(ctx_ref[0, pl.ds(jnp.clip(t - 4, 0, N_CTX - 1), 1), :])

    @pl.when(t == 9)
    def _():
        bcast(eot_ref[0, 0:1, :])

    @pl.when(t >= 10)
    def _():
        bcast(pad_ref[0, pl.ds(jnp.clip(t - 10, 0, PAD_LEN - 1), 1), :])


def kernel(ctx, emb_sot, emb_cls, emb_eot, emb_pad):
    tposed = pl.pallas_call(
        _body,
        grid=(CTX_LEN,),
        in_specs=[
            pl.BlockSpec((1, N_CTX, D), lambda t: (0, 0, 0)),
            pl.BlockSpec((1, 1, D), lambda t: (0, 0, 0)),
            pl.BlockSpec((1, 1, D), lambda t: (0, 0, 0)),
            pl.BlockSpec((1, PAD_LEN, D), lambda t: (0, 0, 0)),
            pl.BlockSpec((CLS_NUM, N_CLS_TOK, D), lambda t: (0, 0, 0)),
        ],
        out_specs=pl.BlockSpec((1, CLS_NUM, D), lambda t: (t, 0, 0)),
        out_shape=jax.ShapeDtypeStruct((CTX_LEN, CLS_NUM, D), jnp.float32),
    )(ctx, emb_sot, emb_eot, emb_pad, emb_cls)
    return jnp.transpose(tposed, (1, 0, 2))


# speedup vs baseline: 11.1235x; 1.1600x over previous
import jax
import jax.numpy as jnp
from jax.experimental import pallas as pl

CLS_NUM = 1000
D = 512
N_PREFIX = 3
N_SUFFIX = 2
N_CTX = N_PREFIX + N_SUFFIX
N_CLS_TOK = 3
CTX_LEN = 77
PAD_SIZE = CTX_LEN - (N_CTX + N_CLS_TOK + 2)
PAD_LEN = 75

SLAB = 7
NSTEPS = CTX_LEN // SLAB


def _body(ctx_ref, sot_ref, eot_ref, pad_ref, cls_ref, out_ref):
    s = pl.program_id(0)

    for r in range(SLAB):
        t = s * SLAB + r

        def bcast(row, r=r):
            out_ref[r, :, :] = jnp.broadcast_to(row.reshape(1, D),
                                                (CLS_NUM, D))

        @pl.when(t == 0)
        def _(r=r):
            bcast(sot_ref[0, 0:1, :])

        @pl.when(jnp.logical_and(t >= 1, t < 4))
        def _(r=r, t=t):
            bcast(ctx_ref[0, pl.ds(jnp.clip(t - 1, 0, N_CTX - 1), 1), :])

        @pl.when(jnp.logical_and(t >= 4, t < 7))
        def _(r=r, t=t):
            j = jnp.clip(t - 4, 0, N_CLS_TOK - 1)
            out_ref[r, :, :] = cls_ref[:, pl.ds(j, 1), :].reshape(CLS_NUM, D)

        @pl.when(jnp.logical_and(t >= 7, t < 9))
        def _(r=r, t=t):
            bcast(ctx_ref[0, pl.ds(jnp.clip(t - 4, 0, N_CTX - 1), 1), :])

        @pl.when(t == 9)
        def _(r=r):
            bcast(eot_ref[0, 0:1, :])

        @pl.when(t >= 10)
        def _(r=r, t=t):
            bcast(pad_ref[0, pl.ds(jnp.clip(t - 10, 0, PAD_LEN - 1), 1), :])


def kernel(ctx, emb_sot, emb_cls, emb_eot, emb_pad):
    tposed = pl.pallas_call(
        _body,
        grid=(NSTEPS,),
        in_specs=[
            pl.BlockSpec((1, N_CTX, D), lambda t: (0, 0, 0)),
            pl.BlockSpec((1, 1, D), lambda t: (0, 0, 0)),
            pl.BlockSpec((1, 1, D), lambda t: (0, 0, 0)),
            pl.BlockSpec((1, PAD_LEN, D), lambda t: (0, 0, 0)),
            pl.BlockSpec((CLS_NUM, N_CLS_TOK, D), lambda t: (0, 0, 0)),
        ],
        out_specs=pl.BlockSpec((SLAB, CLS_NUM, D), lambda t: (t, 0, 0)),
        out_shape=jax.ShapeDtypeStruct((CTX_LEN, CLS_NUM, D), jnp.float32),
    )(ctx, emb_sot, emb_eot, emb_pad, emb_cls)
    return jnp.transpose(tposed, (1, 0, 2))
